# baseline (speedup 1.0000x reference)
import functools

import jax
import jax.numpy as jnp
from jax import lax
from jax.experimental import pallas as pl
from jax.experimental.pallas import tpu as pltpu
from jax.experimental.pallas import tpu_sc as plsc

NC = 2
NS = 16
NT = NC * NS
LANES = 16
SUB = 128
NSUB = 16
CHUNK = SUB * NSUB


def _spmem_rows(n_nodes):
    main = (-(-n_nodes // NS) + 7) // 8 * 8
    last = n_nodes - main * (NS - 1)
    assert 0 < last <= main
    return main, last


def _main_kernel(n_nodes, n_edges, ksize):
    nblocks = n_edges // CHUNK
    assert nblocks * CHUNK == n_edges
    rows_main, rows_last = _spmem_rows(n_nodes)
    kmax = float(ksize - 1)

    mesh = plsc.VectorSubcoreMesh(
        core_axis_name="c", subcore_axis_name="s", num_cores=NC, num_subcores=NS
    )

    vmem_f = lambda n: pltpu.VMEM((n,), jnp.float32)
    vmem_i2 = lambda: pltpu.VMEM((NSUB, SUB), jnp.int32)

    @functools.partial(
        pl.kernel,
        mesh=mesh,
        out_type=tuple(jax.ShapeDtypeStruct((n_nodes,), jnp.float32)
                       for _ in range(4)),
        compiler_params=pltpu.CompilerParams(needs_layout_passes=False, use_tc_tiling_on_sc=False),
        scratch_types=[
            pltpu.VMEM_SHARED((n_nodes, 2), jnp.float32),
            pltpu.VMEM_SHARED((n_nodes,), jnp.float32),
            pltpu.VMEM_SHARED((n_nodes,), jnp.float32),
            [vmem_i2(), vmem_i2()],
            [pltpu.VMEM((CHUNK,), jnp.int32),
             pltpu.VMEM((CHUNK,), jnp.int32)],
            [vmem_f(CHUNK), vmem_f(CHUNK)],
            [pltpu.VMEM((CHUNK, 2), jnp.float32),
             pltpu.VMEM((CHUNK, 2), jnp.float32)],
            [vmem_f(CHUNK), vmem_f(CHUNK)],
            vmem_f(SUB),
            vmem_f(64),
            [pltpu.SemaphoreType.DMA, pltpu.SemaphoreType.DMA],
            [pltpu.SemaphoreType.DMA, pltpu.SemaphoreType.DMA],
            [pltpu.SemaphoreType.DMA, pltpu.SemaphoreType.DMA],
            [pltpu.SemaphoreType.DMA, pltpu.SemaphoreType.DMA],
            [pltpu.SemaphoreType.DMA, pltpu.SemaphoreType.DMA],
        ],
    )
    def k(x_h, row_h, col_h, u_h, wt_h, m0_h, d0_h, m1_h, d1_h,
          xsp2, asp_m, asp_d, row_v, col_v, u_v, xg2_v,
          msg_v, ones_v, wt_v, sem_r, sem_c, sem_u, sem_g, sem_s):
        c = lax.axis_index("c")
        s = lax.axis_index("s")
        wid = c * NS + s
        zi = jnp.zeros((LANES,), jnp.int32)
        oi = jnp.ones((LANES,), jnp.int32)
        iota = lax.iota(jnp.int32, LANES)
        zf = jnp.zeros((LANES,), jnp.float32)
        of = jnp.ones((LANES,), jnp.float32)

        pltpu.sync_copy(wt_h, wt_v)

        def zbody(j, _):
            msg_v[0][pl.ds(j * LANES, LANES)] = zf
            return 0
        lax.fori_loop(0, CHUNK // LANES, zbody, 0)
        for j in range(SUB // LANES):
            ones_v[pl.ds(j * LANES, LANES)] = of

        def init_tile(nrows):
            base = s * rows_main
            done = 0
            while done < nrows:
                step = min(CHUNK, nrows - done)
                sl_sp = pl.ds(base + done, step)
                sl_v = pl.ds(0, step)
                pltpu.sync_copy(msg_v[0].at[sl_v], asp_m.at[sl_sp])
                pltpu.sync_copy(msg_v[0].at[sl_v], asp_d.at[sl_sp])
                pltpu.sync_copy(x_h.at[sl_sp], xg2_v[0].at[sl_v])
                pltpu.sync_copy(xg2_v[0].at[sl_v], xsp2.at[sl_sp])
                done += step

        pl.when(s < NS - 1)(lambda: init_tile(rows_main))
        pl.when(s == NS - 1)(lambda: init_tile(rows_last))
        plsc.subcore_barrier()

        def issue_lin_c(p, blk):
            pltpu.async_copy(col_h.at[pl.ds(blk * CHUNK, CHUNK)],
                             col_v[p], sem_c[p])

        def issue_lin_u(p, blk):
            pltpu.async_copy(u_h.at[pl.ds(blk * CHUNK, CHUNK)],
                             u_v[p], sem_u[p])

        def issue_lin_r(p, blk):
            pltpu.async_copy(row_h.at[pl.ds(blk * NSUB, NSUB)],
                             row_v[p], sem_r[p])

        def wait_lin_r(p):
            pltpu.make_async_copy(row_h.at[pl.ds(0, NSUB)],
                                  row_v[p], sem_r[p]).wait()

        def wait_lin_c(p):
            pltpu.make_async_copy(col_h.at[pl.ds(0, CHUNK)],
                                  col_v[p], sem_c[p]).wait()

        def wait_lin_u(p):
            pltpu.make_async_copy(u_h.at[pl.ds(0, CHUNK)],
                                  u_v[p], sem_u[p]).wait()

        def gather_descs(p):
            return [(xsp2.at[col_v[p]], xg2_v[p])]

        def issue_g(p):
            for src, dst in gather_descs(p):
                pltpu.async_copy(src, dst, sem_g[p])

        def wait_g(p):
            for src, dst in gather_descs(p):
                pltpu.make_async_copy(src, dst, sem_g[p]).wait()

        def scatter_descs(p):
            ds_ = []
            for k2 in range(NSUB):
                idx = row_v[p].at[k2]
                src = pl.ds(k2 * SUB, SUB)
                ds_.append((msg_v[p].at[src], asp_m.at[idx]))
                ds_.append((ones_v, asp_d.at[idx]))
            return ds_

        def issue_s(p):
            for src, dst in scatter_descs(p):
                pltpu.async_copy(src, dst, sem_s[p], add=True)

        def wait_s(p):
            for src, dst in scatter_descs(p):
                pltpu.make_async_copy(src, dst, sem_s[p]).wait()

        def cbody(p):
            def body(j):
                off = j * LANES
                uu = u_v[p][pl.ds(off, LANES)]
                v = uu * kmax
                i0 = v.astype(jnp.int32)
                i1 = i0 + 1
                frac = v - i0.astype(jnp.float32)
                w00 = plsc.load_gather(wt_v, [i0])
                w01 = plsc.load_gather(wt_v, [i1])
                w10 = plsc.load_gather(wt_v, [i0 + 32])
                w11 = plsc.load_gather(wt_v, [i1 + 32])
                we0 = w00 + frac * (w01 - w00)
                we1 = w10 + frac * (w11 - w10)
                lane = off + iota
                xs0 = plsc.load_gather(xg2_v[p], [lane, zi])
                xs1 = plsc.load_gather(xg2_v[p], [lane, oi])
                msg_v[p][pl.ds(off, LANES)] = xs0 * we0 + xs1 * we1
            return body

        nloc = (nblocks - wid + NT - 1) // NT
        npairs = (nloc + 1) // 2

        def half(p, cur, has_cur, not_first):
            nxt = cur + NT
            nxt2 = cur + 2 * NT
            lim = wid + nloc * NT
            has_nxt = nxt < lim
            has_nxt2 = nxt2 < lim

            @pl.when(has_cur)
            def _():
                wait_lin_u(p)
                wait_g(p)
                pl.when(has_nxt2)(lambda: issue_lin_c(p, nxt2))

                @pl.when(has_nxt)
                def _():
                    wait_lin_c(1 - p)
                    issue_g(1 - p)
                    issue_lin_u(1 - p, nxt)

                plsc.parallel_loop(0, CHUNK // LANES, unroll=4)(cbody(p))
                pl.when(not_first)(lambda: wait_s(1 - p))
                wait_lin_r(p)
                issue_s(p)
                pl.when(has_nxt)(lambda: issue_lin_r(1 - p, nxt))

        issue_lin_c(0, wid)
        issue_lin_u(0, wid)
        issue_lin_r(0, wid)
        wait_lin_c(0)
        issue_g(0)
        pl.when(nloc > 1)(lambda: issue_lin_c(1, wid + NT))

        def pair_body(q, _):
            blk_a = wid + (2 * q) * NT
            blk_b = blk_a + NT
            half(0, blk_a, True, q > 0)
            half(1, blk_b, blk_b < (wid + nloc * NT), True)
            return 0

        lax.fori_loop(0, npairs, pair_body, 0)

        pl.when(lax.rem(nloc, 2) == 1)(lambda: wait_s(0))
        pl.when(lax.rem(nloc, 2) == 0)(lambda: wait_s(1))
        plsc.subcore_barrier()

        def writeback(m_h, d_h, nrows):
            base = s * rows_main
            done = 0
            while done < nrows:
                step = min(CHUNK, nrows - done)
                sl_sp = pl.ds(base + done, step)
                sl_v = pl.ds(0, step)
                pltpu.sync_copy(asp_m.at[sl_sp], msg_v[0].at[sl_v])
                pltpu.sync_copy(msg_v[0].at[sl_v], m_h.at[sl_sp])
                pltpu.sync_copy(asp_d.at[sl_sp], u_v[0].at[sl_v])
                pltpu.sync_copy(u_v[0].at[sl_v], d_h.at[sl_sp])
                done += step

        for ci, (m_h, d_h) in enumerate(((m0_h, d0_h), (m1_h, d1_h))):
            pl.when((c == ci) & (s < NS - 1))(
                functools.partial(writeback, m_h, d_h, rows_main))
            pl.when((c == ci) & (s == NS - 1))(
                functools.partial(writeback, m_h, d_h, rows_last))

    return k


def _combine_kernel(n_nodes):
    rows_main = (-(-n_nodes // NT) + 15) // 16 * 16
    rows_last = n_nodes - rows_main * (NT - 1)
    assert 0 < rows_last <= rows_main and rows_last % LANES == 0

    mesh = plsc.VectorSubcoreMesh(
        core_axis_name="c", subcore_axis_name="s", num_cores=NC, num_subcores=NS
    )

    @functools.partial(
        pl.kernel,
        mesh=mesh,
        out_type=jax.ShapeDtypeStruct((n_nodes,), jnp.float32),
        compiler_params=pltpu.CompilerParams(needs_layout_passes=False),
        scratch_types=[
            pltpu.VMEM((rows_main,), jnp.float32),
            pltpu.VMEM((rows_main,), jnp.float32),
            pltpu.VMEM((rows_main,), jnp.float32),
            pltpu.VMEM((rows_main,), jnp.float32),
            pltpu.VMEM((rows_main,), jnp.float32),
        ],
    )
    def k(m0_h, d0_h, m1_h, d1_h, out_h, m0_v, d0_v, m1_v, d1_v, o_v):
        c = lax.axis_index("c")
        s = lax.axis_index("s")
        wid = c * NS + s
        base = wid * rows_main

        def run(sz):
            pltpu.sync_copy(m0_h.at[pl.ds(base, sz)], m0_v.at[pl.ds(0, sz)])
            pltpu.sync_copy(d0_h.at[pl.ds(base, sz)], d0_v.at[pl.ds(0, sz)])
            pltpu.sync_copy(m1_h.at[pl.ds(base, sz)], m1_v.at[pl.ds(0, sz)])
            pltpu.sync_copy(d1_h.at[pl.ds(base, sz)], d1_v.at[pl.ds(0, sz)])

            def cbody(j, _):
                off = j * LANES
                m = m0_v[pl.ds(off, LANES)] + m1_v[pl.ds(off, LANES)]
                dg = d0_v[pl.ds(off, LANES)] + d1_v[pl.ds(off, LANES)]
                o_v[pl.ds(off, LANES)] = m / jnp.maximum(dg, 1.0)
                return 0
            lax.fori_loop(0, sz // LANES, cbody, 0)
            pltpu.sync_copy(o_v.at[pl.ds(0, sz)], out_h.at[pl.ds(base, sz)])

        pl.when(wid < NT - 1)(lambda: run(rows_main))
        pl.when(wid == NT - 1)(lambda: run(rows_last))

    return k


def kernel(x, edge_index, pseudo, weight):
    n_nodes, in_ch = x.shape
    n_edges = edge_index.shape[1]
    ksize = weight.shape[0]
    assert in_ch == 2 and weight.shape[2] == 1

    row = edge_index[0].reshape(n_edges // SUB, SUB)
    col = edge_index[1]
    u = pseudo[:, 0]
    wt = jnp.zeros((64,), jnp.float32)
    wt = wt.at[:ksize].set(weight[:, 0, 0]).at[32:32 + ksize].set(weight[:, 1, 0])

    m0, d0, m1, d1 = _main_kernel(n_nodes, n_edges, ksize)(
        x, row, col, u, wt)
    out = _combine_kernel(n_nodes)(m0, d0, m1, d1)
    return out.reshape(n_nodes, 1)


# speedup vs baseline: 1.4435x; 1.4435x over previous
import functools

import jax
import jax.numpy as jnp
from jax import lax
from jax.experimental import pallas as pl
from jax.experimental.pallas import tpu as pltpu
from jax.experimental.pallas import tpu_sc as plsc

NC = 2
NS = 16
NT = NC * NS
LANES = 16
SUB = 128
NSUB = 16
CHUNK = SUB * NSUB


def _spmem_rows(n_nodes):
    main = (-(-n_nodes // NS) + 7) // 8 * 8
    last = n_nodes - main * (NS - 1)
    assert 0 < last <= main
    return main, last


def _main_kernel(n_nodes, n_edges, ksize):
    nblocks = n_edges // CHUNK
    assert nblocks * CHUNK == n_edges
    rows_main, rows_last = _spmem_rows(n_nodes)
    kmax = float(ksize - 1)

    mesh = plsc.VectorSubcoreMesh(
        core_axis_name="c", subcore_axis_name="s", num_cores=NC, num_subcores=NS
    )

    vmem_f = lambda n: pltpu.VMEM((n,), jnp.float32)
    vmem_i2 = lambda: pltpu.VMEM((NSUB, SUB), jnp.int32)

    @functools.partial(
        pl.kernel,
        mesh=mesh,
        out_type=tuple(jax.ShapeDtypeStruct((n_nodes,), jnp.float32)
                       for _ in range(4)),
        compiler_params=pltpu.CompilerParams(needs_layout_passes=False),
        scratch_types=[
            pltpu.VMEM_SHARED((n_nodes,), jnp.float32),
            pltpu.VMEM_SHARED((n_nodes,), jnp.float32),
            pltpu.VMEM_SHARED((n_nodes,), jnp.float32),
            pltpu.VMEM_SHARED((n_nodes,), jnp.float32),
            [vmem_i2(), vmem_i2()],
            [vmem_i2(), vmem_i2()],
            [vmem_f(CHUNK), vmem_f(CHUNK)],
            [vmem_f(CHUNK), vmem_f(CHUNK)],
            [vmem_f(CHUNK), vmem_f(CHUNK)],
            [vmem_f(CHUNK), vmem_f(CHUNK)],
            vmem_f(SUB),
            vmem_f(64),
            [pltpu.SemaphoreType.DMA, pltpu.SemaphoreType.DMA],
            [pltpu.SemaphoreType.DMA, pltpu.SemaphoreType.DMA],
            [pltpu.SemaphoreType.DMA, pltpu.SemaphoreType.DMA],
            [pltpu.SemaphoreType.DMA, pltpu.SemaphoreType.DMA],
            [pltpu.SemaphoreType.DMA, pltpu.SemaphoreType.DMA],
        ],
    )
    def k(x0_h, x1_h, row_h, col_h, u_h, wt_h, m0_h, d0_h, m1_h, d1_h,
          xsp0, xsp1, asp_m, asp_d, row_v, col_v, u_v, xg0_v, xg1_v,
          msg_v, ones_v, wt_v, sem_r, sem_c, sem_u, sem_g, sem_s):
        c = lax.axis_index("c")
        s = lax.axis_index("s")
        wid = c * NS + s
        zf = jnp.zeros((LANES,), jnp.float32)
        of = jnp.ones((LANES,), jnp.float32)

        pltpu.sync_copy(wt_h, wt_v)

        def zbody(j, _):
            msg_v[0][pl.ds(j * LANES, LANES)] = zf
            return 0
        lax.fori_loop(0, CHUNK // LANES, zbody, 0)
        for j in range(SUB // LANES):
            ones_v[pl.ds(j * LANES, LANES)] = of

        def init_tile(nrows):
            base = s * rows_main
            done = 0
            while done < nrows:
                step = min(CHUNK, nrows - done)
                sl_sp = pl.ds(base + done, step)
                sl_v = pl.ds(0, step)
                pltpu.sync_copy(msg_v[0].at[sl_v], asp_m.at[sl_sp])
                pltpu.sync_copy(msg_v[0].at[sl_v], asp_d.at[sl_sp])
                pltpu.sync_copy(x0_h.at[sl_sp], xg0_v[0].at[sl_v])
                pltpu.sync_copy(xg0_v[0].at[sl_v], xsp0.at[sl_sp])
                pltpu.sync_copy(x1_h.at[sl_sp], xg1_v[0].at[sl_v])
                pltpu.sync_copy(xg1_v[0].at[sl_v], xsp1.at[sl_sp])
                done += step

        pl.when(s < NS - 1)(lambda: init_tile(rows_main))
        pl.when(s == NS - 1)(lambda: init_tile(rows_last))
        plsc.subcore_barrier()

        def issue_lin_c(p, blk):
            pltpu.async_copy(col_h.at[pl.ds(blk * NSUB, NSUB)],
                             col_v[p], sem_c[p])

        def issue_lin_u(p, blk):
            pltpu.async_copy(u_h.at[pl.ds(blk * CHUNK, CHUNK)],
                             u_v[p], sem_u[p])

        def issue_lin_r(p, blk):
            pltpu.async_copy(row_h.at[pl.ds(blk * NSUB, NSUB)],
                             row_v[p], sem_r[p])

        def wait_lin_r(p):
            pltpu.make_async_copy(row_h.at[pl.ds(0, NSUB)],
                                  row_v[p], sem_r[p]).wait()

        def wait_lin_c(p):
            pltpu.make_async_copy(col_h.at[pl.ds(0, NSUB)],
                                  col_v[p], sem_c[p]).wait()

        def wait_lin_u(p):
            pltpu.make_async_copy(u_h.at[pl.ds(0, CHUNK)],
                                  u_v[p], sem_u[p]).wait()

        def gather_descs(p):
            ds_ = []
            for k2 in range(NSUB):
                idx = col_v[p].at[k2]
                dst = pl.ds(k2 * SUB, SUB)
                ds_.append((xsp0.at[idx], xg0_v[p].at[dst]))
                ds_.append((xsp1.at[idx], xg1_v[p].at[dst]))
            return ds_

        def issue_g(p):
            for src, dst in gather_descs(p):
                pltpu.async_copy(src, dst, sem_g[p])

        def wait_g(p):
            for src, dst in gather_descs(p):
                pltpu.make_async_copy(src, dst, sem_g[p]).wait()

        def scatter_descs(p):
            ds_ = []
            for k2 in range(NSUB):
                idx = row_v[p].at[k2]
                src = pl.ds(k2 * SUB, SUB)
                ds_.append((msg_v[p].at[src], asp_m.at[idx]))
                ds_.append((ones_v, asp_d.at[idx]))
            return ds_

        def issue_s(p):
            for src, dst in scatter_descs(p):
                pltpu.async_copy(src, dst, sem_s[p], add=True)

        def wait_s(p):
            for src, dst in scatter_descs(p):
                pltpu.make_async_copy(src, dst, sem_s[p]).wait()

        def cbody(p):
            def body(j):
                off = j * LANES
                uu = u_v[p][pl.ds(off, LANES)]
                v = uu * kmax
                i0 = v.astype(jnp.int32)
                i1 = i0 + 1
                frac = v - i0.astype(jnp.float32)
                w00 = plsc.load_gather(wt_v, [i0])
                w01 = plsc.load_gather(wt_v, [i1])
                w10 = plsc.load_gather(wt_v, [i0 + 32])
                w11 = plsc.load_gather(wt_v, [i1 + 32])
                we0 = w00 + frac * (w01 - w00)
                we1 = w10 + frac * (w11 - w10)
                xs0 = xg0_v[p][pl.ds(off, LANES)]
                xs1 = xg1_v[p][pl.ds(off, LANES)]
                msg_v[p][pl.ds(off, LANES)] = xs0 * we0 + xs1 * we1
            return body

        nloc = (nblocks - wid + NT - 1) // NT
        npairs = (nloc + 1) // 2

        def half(p, cur, has_cur, not_first):
            nxt = cur + NT
            nxt2 = cur + 2 * NT
            lim = wid + nloc * NT
            has_nxt = nxt < lim
            has_nxt2 = nxt2 < lim

            @pl.when(has_cur)
            def _():
                wait_lin_u(p)
                wait_g(p)
                pl.when(has_nxt2)(lambda: issue_lin_c(p, nxt2))

                @pl.when(has_nxt)
                def _():
                    wait_lin_c(1 - p)
                    issue_g(1 - p)
                    issue_lin_u(1 - p, nxt)

                plsc.parallel_loop(0, CHUNK // LANES, unroll=4)(cbody(p))
                pl.when(not_first)(lambda: wait_s(1 - p))
                wait_lin_r(p)
                issue_s(p)
                pl.when(has_nxt)(lambda: issue_lin_r(1 - p, nxt))

        issue_lin_c(0, wid)
        issue_lin_u(0, wid)
        issue_lin_r(0, wid)
        wait_lin_c(0)
        issue_g(0)
        pl.when(nloc > 1)(lambda: issue_lin_c(1, wid + NT))

        def pair_body(q, _):
            blk_a = wid + (2 * q) * NT
            blk_b = blk_a + NT
            half(0, blk_a, True, q > 0)
            half(1, blk_b, blk_b < (wid + nloc * NT), True)
            return 0

        lax.fori_loop(0, npairs, pair_body, 0)

        pl.when(lax.rem(nloc, 2) == 1)(lambda: wait_s(0))
        pl.when(lax.rem(nloc, 2) == 0)(lambda: wait_s(1))
        plsc.subcore_barrier()

        def writeback(m_h, d_h, nrows):
            base = s * rows_main
            done = 0
            while done < nrows:
                step = min(CHUNK, nrows - done)
                sl_sp = pl.ds(base + done, step)
                sl_v = pl.ds(0, step)
                pltpu.sync_copy(asp_m.at[sl_sp], msg_v[0].at[sl_v])
                pltpu.sync_copy(msg_v[0].at[sl_v], m_h.at[sl_sp])
                pltpu.sync_copy(asp_d.at[sl_sp], u_v[0].at[sl_v])
                pltpu.sync_copy(u_v[0].at[sl_v], d_h.at[sl_sp])
                done += step

        for ci, (m_h, d_h) in enumerate(((m0_h, d0_h), (m1_h, d1_h))):
            pl.when((c == ci) & (s < NS - 1))(
                functools.partial(writeback, m_h, d_h, rows_main))
            pl.when((c == ci) & (s == NS - 1))(
                functools.partial(writeback, m_h, d_h, rows_last))

    return k


def _combine_kernel(n_nodes):
    rows_main = (-(-n_nodes // NT) + 15) // 16 * 16
    rows_last = n_nodes - rows_main * (NT - 1)
    assert 0 < rows_last <= rows_main and rows_last % LANES == 0

    mesh = plsc.VectorSubcoreMesh(
        core_axis_name="c", subcore_axis_name="s", num_cores=NC, num_subcores=NS
    )

    @functools.partial(
        pl.kernel,
        mesh=mesh,
        out_type=jax.ShapeDtypeStruct((n_nodes,), jnp.float32),
        compiler_params=pltpu.CompilerParams(needs_layout_passes=False),
        scratch_types=[
            pltpu.VMEM((rows_main,), jnp.float32),
            pltpu.VMEM((rows_main,), jnp.float32),
            pltpu.VMEM((rows_main,), jnp.float32),
            pltpu.VMEM((rows_main,), jnp.float32),
            pltpu.VMEM((rows_main,), jnp.float32),
        ],
    )
    def k(m0_h, d0_h, m1_h, d1_h, out_h, m0_v, d0_v, m1_v, d1_v, o_v):
        c = lax.axis_index("c")
        s = lax.axis_index("s")
        wid = c * NS + s
        base = wid * rows_main

        def run(sz):
            pltpu.sync_copy(m0_h.at[pl.ds(base, sz)], m0_v.at[pl.ds(0, sz)])
            pltpu.sync_copy(d0_h.at[pl.ds(base, sz)], d0_v.at[pl.ds(0, sz)])
            pltpu.sync_copy(m1_h.at[pl.ds(base, sz)], m1_v.at[pl.ds(0, sz)])
            pltpu.sync_copy(d1_h.at[pl.ds(base, sz)], d1_v.at[pl.ds(0, sz)])

            def cbody(j, _):
                off = j * LANES
                m = m0_v[pl.ds(off, LANES)] + m1_v[pl.ds(off, LANES)]
                dg = d0_v[pl.ds(off, LANES)] + d1_v[pl.ds(off, LANES)]
                o_v[pl.ds(off, LANES)] = m / jnp.maximum(dg, 1.0)
                return 0
            lax.fori_loop(0, sz // LANES, cbody, 0)
            pltpu.sync_copy(o_v.at[pl.ds(0, sz)], out_h.at[pl.ds(base, sz)])

        pl.when(wid < NT - 1)(lambda: run(rows_main))
        pl.when(wid == NT - 1)(lambda: run(rows_last))

    return k


def kernel(x, edge_index, pseudo, weight):
    n_nodes, in_ch = x.shape
    n_edges = edge_index.shape[1]
    ksize = weight.shape[0]
    assert in_ch == 2 and weight.shape[2] == 1

    x0 = x[:, 0]
    x1 = x[:, 1]
    row = edge_index[0].reshape(n_edges // SUB, SUB)
    col = edge_index[1].reshape(n_edges // SUB, SUB)
    u = pseudo[:, 0]
    wt = jnp.zeros((64,), jnp.float32)
    wt = wt.at[:ksize].set(weight[:, 0, 0]).at[32:32 + ksize].set(weight[:, 1, 0])

    m0, d0, m1, d1 = _main_kernel(n_nodes, n_edges, ksize)(
        x0, x1, row, col, u, wt)
    out = _combine_kernel(n_nodes)(m0, d0, m1, d1)
    return out.reshape(n_nodes, 1)
